# Initial kernel scaffold; baseline (speedup 1.0000x reference)
import jax
import jax.numpy as jnp
from jax.experimental import pallas as pl


def kernel(x, enc, codebook, dec):
    raise NotImplementedError("write your pallas kernel here")


# speedup vs baseline: 1.3341x; 1.3341x over previous
import functools

import jax
import jax.numpy as jnp
from jax import lax
from jax.experimental import pallas as pl
from jax.experimental.pallas import tpu as pltpu
from jax.experimental.pallas import tpu_sc as plsc

B, N = 64, 1024
TOK = B * N
IN_DIM, HID, EDIM, NCODES = 256, 256, 32, 8192

TILE = 512
NT = TOK // TILE
CHUNK = 2048
NCHUNK = NCODES // CHUNK

NC, NS = 2, 16
NW = NC * NS
BPW = TOK // NW
GCH = 128
NGC = BPW // GCH


def _dot(a, b):
    return lax.dot_general(a, b, (((1,), (0,)), ((), ())),
                           preferred_element_type=jnp.float32)


def _dot_t(a, b):
    return lax.dot_general(a, b, (((1,), (1,)), ((), ())),
                           preferred_element_type=jnp.float32)


def _ln(h, g, b):
    mu = jnp.mean(h, axis=-1, keepdims=True)
    var = jnp.mean((h - mu) ** 2, axis=-1, keepdims=True)
    return (h - mu) / jnp.sqrt(var + 1e-5) * g + b


def _encvq_body(x_ref,
                w1, b1, g1, lb1, w2, b2, g2, lb2,
                w3, b3, g3, lb3, w4, b4, g4, lb4,
                cb0_ref, cs_ref,
                e_ref, idx_ref):
    h = x_ref[...]
    h = jax.nn.gelu(_ln(_dot(h, w1[...]) + b1[...], g1[...], lb1[...]))
    h = jax.nn.gelu(_ln(_dot(h, w2[...]) + b2[...], g2[...], lb2[...]))
    h = jax.nn.gelu(_ln(_dot(h, w3[...]) + b3[...], g3[...], lb3[...]))
    e = _ln(_dot(h, w4[...]) + b4[...], g4[...], lb4[...])
    e_ref[...] = e

    eb = e.astype(jnp.bfloat16)
    es = jnp.sum(e * e, axis=1, keepdims=True)

    run_min = jnp.full((TILE, 1), jnp.inf, jnp.float32)
    run_idx = jnp.zeros((TILE, 1), jnp.int32)
    for c in range(NCHUNK):
        sl = pl.ds(c * CHUNK, CHUNK)
        p = _dot_t(eb, cb0_ref[sl, :])
        d = (es - 2.0 * p) + cs_ref[:, sl]
        m = jnp.min(d, axis=1, keepdims=True)
        io = lax.broadcasted_iota(jnp.int32, (TILE, CHUNK), 1)
        ii = jnp.min(jnp.where(d == m, io, jnp.int32(2 ** 30)),
                     axis=1, keepdims=True) + c * CHUNK
        better = m < run_min
        run_idx = jnp.where(better, ii, run_idx)
        run_min = jnp.where(better, m, run_min)
    idx_ref[...] = run_idx


def _dec_body(q_ref, e_ref,
              w1, b1, g1, lb1, w2, b2, g2, lb2,
              w3, b3, g3, lb3, w4, b4,
              out_ref, loss_ref):
    q = q_ref[...]
    e = e_ref[...]

    @pl.when(pl.program_id(0) == 0)
    def _init():
        loss_ref[...] = jnp.zeros((1, 1), jnp.float32)

    df = e - q
    loss_ref[...] += jnp.sum(df * df).reshape(1, 1)

    qst = e + (q - e)
    h = jax.nn.gelu(_ln(_dot(qst, w1[...]) + b1[...], g1[...], lb1[...]))
    h = jax.nn.gelu(_ln(_dot(h, w2[...]) + b2[...], g2[...], lb2[...]))
    h = jax.nn.gelu(_ln(_dot(h, w3[...]) + b3[...], g3[...], lb3[...]))
    out_ref[...] = _dot(h, w4[...]) + b4[...]


def _full(shape):
    return pl.BlockSpec(shape, lambda i: (0,) * len(shape))


def _encvq_call(xf, enc_flat, cb0, cs):
    in_specs = ([pl.BlockSpec((TILE, IN_DIM), lambda i: (i, 0))]
                + [_full(a.shape) for a in enc_flat]
                + [_full((NCODES, EDIM))] + [_full((1, NCODES))])
    return pl.pallas_call(
        _encvq_body,
        grid=(NT,),
        in_specs=in_specs,
        out_specs=[pl.BlockSpec((TILE, EDIM), lambda i: (i, 0)),
                   pl.BlockSpec((TILE, 1), lambda i: (i, 0))],
        out_shape=[jax.ShapeDtypeStruct((TOK, EDIM), jnp.float32),
                   jax.ShapeDtypeStruct((TOK, 1), jnp.int32)],
    )(xf, *enc_flat, cb0, cs)


def _dec_call(q, e, dec_flat):
    in_specs = ([pl.BlockSpec((TILE, EDIM), lambda i: (i, 0)),
                 pl.BlockSpec((TILE, EDIM), lambda i: (i, 0))]
                + [_full(a.shape) for a in dec_flat])
    return pl.pallas_call(
        _dec_body,
        grid=(NT,),
        in_specs=in_specs,
        out_specs=[pl.BlockSpec((TILE, IN_DIM), lambda i: (i, 0)),
                   pl.BlockSpec((1, 1), lambda i: (0, 0))],
        out_shape=[jax.ShapeDtypeStruct((TOK, IN_DIM), jnp.float32),
                   jax.ShapeDtypeStruct((1, 1), jnp.float32)],
    )(q, e, *dec_flat)


def _sc_gather(codebook, idx3):
    mesh = plsc.VectorSubcoreMesh(core_axis_name="c", subcore_axis_name="s")

    @functools.partial(
        pl.kernel, mesh=mesh,
        compiler_params=pltpu.CompilerParams(use_tc_tiling_on_sc=False),
        out_type=jax.ShapeDtypeStruct((TOK, EDIM), jnp.float32),
        scratch_types=[pltpu.VMEM((NGC, GCH), jnp.int32),
                       pltpu.VMEM((BPW, EDIM), jnp.float32),
                       pltpu.SemaphoreType.DMA])
    def k(table_hbm, idx_hbm, out_hbm, idx_v, rows_v, sem):
        wid = lax.axis_index("s") * NC + lax.axis_index("c")
        pltpu.sync_copy(idx_hbm.at[wid], idx_v)
        cps = []
        for j in range(NGC):
            cps.append(pltpu.async_copy(
                table_hbm.at[idx_v.at[j]],
                rows_v.at[pl.ds(j * GCH, GCH)], sem))
        for c in cps:
            c.wait()
        pltpu.sync_copy(rows_v, out_hbm.at[pl.ds(wid * BPW, BPW)])

    return k(codebook, idx3)


def kernel(x, enc, codebook, dec):
    xf = x.reshape(TOK, IN_DIM)
    row = lambda v: v.reshape(1, -1)
    enc_flat = [enc["fc1"]["w"], row(enc["fc1"]["b"]), row(enc["ln1"]["g"]), row(enc["ln1"]["b"]),
                enc["fc2"]["w"], row(enc["fc2"]["b"]), row(enc["ln2"]["g"]), row(enc["ln2"]["b"]),
                enc["fc3"]["w"], row(enc["fc3"]["b"]), row(enc["ln3"]["g"]), row(enc["ln3"]["b"]),
                enc["fc4"]["w"], row(enc["fc4"]["b"]), row(enc["ln4"]["g"]), row(enc["ln4"]["b"])]
    dec_flat = [dec["fc1"]["w"], row(dec["fc1"]["b"]), row(dec["ln1"]["g"]), row(dec["ln1"]["b"]),
                dec["fc2"]["w"], row(dec["fc2"]["b"]), row(dec["ln2"]["g"]), row(dec["ln2"]["b"]),
                dec["fc3"]["w"], row(dec["fc3"]["b"]), row(dec["ln3"]["g"]), row(dec["ln3"]["b"]),
                dec["fc4"]["w"], row(dec["fc4"]["b"])]

    cs = jnp.sum(codebook ** 2, axis=-1).reshape(1, NCODES)

    e, idx = _encvq_call(xf, enc_flat, codebook, cs)

    idx_flat = idx.reshape(TOK)
    q = _sc_gather(codebook, idx_flat.reshape(NW, NGC, GCH))

    recon, sq = _dec_call(q, e, dec_flat)
    commit_loss = (sq[0, 0] / jnp.float32(TOK * EDIM)).reshape(())
    return recon.reshape(B, N, IN_DIM), commit_loss, idx_flat.reshape(B, N)
